# baseline (speedup 1.0000x reference)
import functools
import jax
import jax.numpy as jnp
from jax import lax
from jax.experimental import pallas as pl
from jax.experimental.pallas import tpu as pltpu
from jax.experimental.pallas import tpu_sc as plsc

NC, NS, L = 2, 16, 16
NW = NC * NS
EB = 64


def _edge_pass(src, dst, asrc, adst, h, nheads, nb0):
    n, d = h.shape
    ept2 = 2 * (src.shape[0] // NW)
    nb1 = ept2 // EB - nb0
    ept0, ept1 = nb0 * EB, nb1 * EB
    npad = ((n + 1 + NS - 1) // NS) * NS
    rpt = npad // NS
    hid = d // nheads
    zden = jnp.zeros((rpt, L), jnp.float32)
    zout = jnp.zeros((rpt, d), jnp.float32)

    mesh = plsc.VectorSubcoreMesh(core_axis_name="c", subcore_axis_name="s",
                                  num_cores=NC, num_subcores=NS)

    @functools.partial(
        pl.kernel,
        out_type=(jax.ShapeDtypeStruct((NC, npad, d), jnp.float32),
                  jax.ShapeDtypeStruct((NC, npad, L), jnp.float32)),
        mesh=mesh,
        compiler_params=pltpu.CompilerParams(use_tc_tiling_on_sc=False),
        scratch_types=[
            pltpu.VMEM((3, EB), jnp.int32),
            pltpu.VMEM((3, EB), jnp.int32),
            pltpu.VMEM((3, EB, L), jnp.float32),
            pltpu.VMEM((3, EB, L), jnp.float32),
            pltpu.VMEM((3, EB, L), jnp.float32),
            pltpu.VMEM((3, EB, d), jnp.float32),
            pltpu.VMEM_SHARED((npad, d), jnp.float32),
            pltpu.VMEM_SHARED((npad, L), jnp.float32),
            pltpu.SemaphoreType.DMA,
            pltpu.SemaphoreType.DMA,
            pltpu.SemaphoreType.DMA,
            pltpu.SemaphoreType.DMA,
            pltpu.SemaphoreType.DMA,
            pltpu.SemaphoreType.DMA,
        ],
    )
    def k(src_hbm, dst_hbm, asrc_hbm, adst_hbm, h_hbm, zden_hbm, zout_hbm,
          out_hbm, den_hbm,
          sidx, didx, srow, drow, wbuf, msg, out_acc, den_acc,
          g0, g1, g2, s0, s1, s2):
        gsem = (g0, g1, g2)
        ssem = (s0, s1, s2)
        c = lax.axis_index("c")
        s = lax.axis_index("s")
        r0 = s * rpt
        pltpu.sync_copy(zden_hbm, den_acc.at[pl.ds(r0, rpt)])
        pltpu.sync_copy(zout_hbm, out_acc.at[pl.ds(r0, rpt)])
        plsc.subcore_barrier()
        base = jnp.where(c == 0, s * ept0, NS * ept0 + s * ept1)
        nt_c = jnp.where(c == 0, nb0 // 3, nb1 // 3)
        lanemask = lax.iota(jnp.int32, L) < nheads

        def issue_gather(j, b):
            off = base + j * EB
            pltpu.sync_copy(src_hbm.at[pl.ds(off, EB)], sidx.at[b])
            pltpu.sync_copy(dst_hbm.at[pl.ds(off, EB)], didx.at[b])
            pltpu.async_copy(asrc_hbm.at[sidx.at[b]], srow.at[b], gsem[b])
            pltpu.async_copy(adst_hbm.at[didx.at[b]], drow.at[b], gsem[b])
            pltpu.async_copy(h_hbm.at[sidx.at[b]], msg.at[b], gsem[b])

        def wait_gather(b):
            pltpu.make_async_copy(asrc_hbm.at[sidx.at[b]], srow.at[b], gsem[b]).wait()
            pltpu.make_async_copy(adst_hbm.at[didx.at[b]], drow.at[b], gsem[b]).wait()
            pltpu.make_async_copy(h_hbm.at[sidx.at[b]], msg.at[b], gsem[b]).wait()

        def issue_scatter(b):
            pltpu.async_copy(wbuf.at[b], den_acc.at[didx.at[b]], ssem[b], add=True)
            pltpu.async_copy(msg.at[b], out_acc.at[didx.at[b]], ssem[b], add=True)

        def wait_scatter(b):
            pltpu.make_async_copy(wbuf.at[b], den_acc.at[didx.at[b]], ssem[b]).wait()
            pltpu.make_async_copy(msg.at[b], out_acc.at[didx.at[b]], ssem[b]).wait()

        def compute(b):
            def edge(e, carry):
                ev = srow[b, e] + drow[b, e]
                ev = jnp.maximum(ev, 0.2 * ev)
                wv = jnp.exp(ev)
                wv = jnp.where(lanemask, wv, 0.0)
                wbuf[b, e] = wv
                for v in range(d // L):
                    sc = wv[(v * L) // hid]
                    msg[b, e, pl.ds(v * L, L)] = msg[b, e, pl.ds(v * L, L)] * sc
                return carry
            lax.fori_loop(0, EB, edge, 0, unroll=2)

        def pipestep(j, k_, head=False, issue_next=True):
            wait_gather(k_)
            if not head:
                wait_scatter((k_ + 1) % 3)
            if issue_next:
                issue_gather(j + 1, (k_ + 1) % 3)
            compute(k_)
            issue_scatter(k_)

        issue_gather(0, 0)
        pipestep(0, 0, head=True)
        pipestep(1, 1, head=True)
        pipestep(2, 2)

        def triple(j3, carry):
            for k_ in range(3):
                pipestep(j3 * 3 + k_, k_)
            return carry

        lax.fori_loop(1, nt_c - 1, triple, 0)
        j0 = (nt_c - 1) * 3
        pipestep(j0, 0)
        pipestep(j0 + 1, 1)
        pipestep(j0 + 2, 2, issue_next=False)
        wait_scatter(1)
        wait_scatter(2)
        plsc.subcore_barrier()
        pltpu.sync_copy(out_acc.at[pl.ds(r0, rpt)],
                        out_hbm.at[c, pl.ds(r0, rpt)])
        pltpu.sync_copy(den_acc.at[pl.ds(r0, rpt)],
                        den_hbm.at[c, pl.ds(r0, rpt)])

    return k(src, dst, asrc, adst, h, zden, zout)


def _blk(n):
    for b in (1000, 500, 250, 200, 125, 100, 50, 40, 25, 20, 10, 8, 5, 4, 2, 1):
        if n % b == 0:
            return b
    return n


def _tc_pre(x, W, As, Ad):
    n, _ = x.shape
    dh = W.shape[1]
    blk = _blk(n)

    def body(x_ref, w_ref, a_ref, b_ref, h_ref, s_ref, t_ref):
        hv = jnp.dot(x_ref[...], w_ref[...], preferred_element_type=jnp.float32)
        h_ref[...] = hv
        s_ref[...] = jnp.dot(hv, a_ref[...], preferred_element_type=jnp.float32)
        t_ref[...] = jnp.dot(hv, b_ref[...], preferred_element_type=jnp.float32)

    return pl.pallas_call(
        body,
        grid=(n // blk,),
        in_specs=[pl.BlockSpec((blk, x.shape[1]), lambda i: (i, 0)),
                  pl.BlockSpec(W.shape, lambda i: (0, 0)),
                  pl.BlockSpec(As.shape, lambda i: (0, 0)),
                  pl.BlockSpec(Ad.shape, lambda i: (0, 0))],
        out_specs=[pl.BlockSpec((blk, dh), lambda i: (i, 0)),
                   pl.BlockSpec((blk, L), lambda i: (i, 0)),
                   pl.BlockSpec((blk, L), lambda i: (i, 0))],
        out_shape=[jax.ShapeDtypeStruct((n, dh), jnp.float32),
                   jax.ShapeDtypeStruct((n, L), jnp.float32),
                   jax.ShapeDtypeStruct((n, L), jnp.float32)],
    )(x, W, As, Ad)


def _tc_mid(p0, p1, dn0, dn1, R, b1, W2, As, Ad):
    n, d1 = p0.shape
    d2 = W2.shape[1]
    blk = _blk(n)

    def body(p0_ref, p1_ref, dn0_ref, dn1_ref, r_ref, b_ref, w_ref, a_ref,
             c_ref, h_ref, s_ref, t_ref):
        den = jnp.dot(dn0_ref[...] + dn1_ref[...], r_ref[...],
                      preferred_element_type=jnp.float32)
        hin = (p0_ref[...] + p1_ref[...]) / (den + 1e-16) + b_ref[...]
        hin = jnp.where(hin > 0, hin, jnp.exp(hin) - 1.0)
        h2 = jnp.dot(hin, w_ref[...], preferred_element_type=jnp.float32)
        h_ref[...] = h2
        s_ref[...] = jnp.dot(h2, a_ref[...], preferred_element_type=jnp.float32)
        t_ref[...] = jnp.dot(h2, c_ref[...], preferred_element_type=jnp.float32)

    return pl.pallas_call(
        body,
        grid=(n // blk,),
        in_specs=[pl.BlockSpec((blk, d1), lambda i: (i, 0)),
                  pl.BlockSpec((blk, d1), lambda i: (i, 0)),
                  pl.BlockSpec((blk, L), lambda i: (i, 0)),
                  pl.BlockSpec((blk, L), lambda i: (i, 0)),
                  pl.BlockSpec(R.shape, lambda i: (0, 0)),
                  pl.BlockSpec((1, d1), lambda i: (0, 0)),
                  pl.BlockSpec(W2.shape, lambda i: (0, 0)),
                  pl.BlockSpec(As.shape, lambda i: (0, 0)),
                  pl.BlockSpec(Ad.shape, lambda i: (0, 0))],
        out_specs=[pl.BlockSpec((blk, d2), lambda i: (i, 0)),
                   pl.BlockSpec((blk, L), lambda i: (i, 0)),
                   pl.BlockSpec((blk, L), lambda i: (i, 0))],
        out_shape=[jax.ShapeDtypeStruct((n, d2), jnp.float32),
                   jax.ShapeDtypeStruct((n, L), jnp.float32),
                   jax.ShapeDtypeStruct((n, L), jnp.float32)],
    )(p0, p1, dn0, dn1, R, b1, W2, As, Ad)


def _tc_fin(q0, q1, dn0, dn1, R, b2):
    n, d2 = q0.shape
    blk = _blk(n)

    def body(q0_ref, q1_ref, dn0_ref, dn1_ref, r_ref, b_ref, o_ref):
        den = jnp.dot(dn0_ref[...] + dn1_ref[...], r_ref[...],
                      preferred_element_type=jnp.float32)
        o_ref[...] = (q0_ref[...] + q1_ref[...]) / (den + 1e-16) + b_ref[...]

    return pl.pallas_call(
        body,
        grid=(n // blk,),
        in_specs=[pl.BlockSpec((blk, d2), lambda i: (i, 0)),
                  pl.BlockSpec((blk, d2), lambda i: (i, 0)),
                  pl.BlockSpec((blk, L), lambda i: (i, 0)),
                  pl.BlockSpec((blk, L), lambda i: (i, 0)),
                  pl.BlockSpec(R.shape, lambda i: (0, 0)),
                  pl.BlockSpec((1, d2), lambda i: (0, 0))],
        out_specs=pl.BlockSpec((blk, d2), lambda i: (i, 0)),
        out_shape=jax.ShapeDtypeStruct((n, d2), jnp.float32),
    )(q0, q1, dn0, dn1, R, b2)


def kernel(x, edge_index, W1, att_src1, att_dst1, b1, W2, att_src2, att_dst2, b2):
    n = x.shape[0]
    e = edge_index.shape[1]
    h1, hid1 = att_src1.shape
    d1 = h1 * hid1
    d2 = W2.shape[1]

    nbt = max(-(-e // (NW * EB)), 6)
    nbt = -(-nbt // 3) * 3
    ept = nbt * EB
    pad = ept * NW - e
    npad = ((n + 1 + NS - 1) // NS) * NS
    src = jnp.concatenate([edge_index[0], jnp.zeros((pad,), jnp.int32)])
    dst = jnp.concatenate(
        [edge_index[1], n + (jnp.arange(pad, dtype=jnp.int32) % (npad - n))])

    eye1 = jnp.eye(h1, L, dtype=jnp.float32)
    As1 = (att_src1[:, :, None] * eye1[:, None, :]).reshape(d1, L)
    Ad1 = (att_dst1[:, :, None] * eye1[:, None, :]).reshape(d1, L)
    eye2 = jnp.eye(1, L, dtype=jnp.float32)
    As2 = (att_src2[:, :, None] * eye2[:, None, :]).reshape(d2, L)
    Ad2 = (att_dst2[:, :, None] * eye2[:, None, :]).reshape(d2, L)
    R1 = jnp.repeat(jnp.eye(L, h1, dtype=jnp.float32), d1 // h1, axis=1)
    R2 = jnp.repeat(jnp.eye(L, 1, dtype=jnp.float32), d2, axis=1)

    tot = 2 * nbt

    def _split(frac0):
        return min(max(6, 3 * round(tot * frac0 / 3)), tot - 6)

    ha, s1, t1 = _tc_pre(x, W1, As1, Ad1)
    p, dn = _edge_pass(src, dst, s1, t1, ha, h1, _split(0.40))
    h2, s2, t2 = _tc_mid(p[0, :n], p[1, :n], dn[0, :n], dn[1, :n], R1,
                         b1.reshape(1, d1), W2, As2, Ad2)
    q, dn2 = _edge_pass(src, dst, s2, t2, h2, 1, _split(0.45))
    return _tc_fin(q[0, :n], q[1, :n], dn2[0, :n], dn2[1, :n], R2,
                   b2.reshape(1, d2))


# speedup vs baseline: 1.1701x; 1.1701x over previous
import functools
import jax
import jax.numpy as jnp
from jax import lax
from jax.experimental import pallas as pl
from jax.experimental.pallas import tpu as pltpu
from jax.experimental.pallas import tpu_sc as plsc

NC, NS, L = 2, 16, 16
NW = NC * NS
EB = 64


def _edge_pass(src, dst, asrc, adst, h, nheads, nb0):
    n, d = h.shape
    ept2 = 2 * (src.shape[0] // NW)
    nb1 = ept2 // EB - nb0
    ept0, ept1 = nb0 * EB, nb1 * EB
    npad = ((n + 1 + NS - 1) // NS) * NS
    rpt = npad // NS
    hid = d // nheads
    zden = jnp.zeros((rpt, L), jnp.float32)
    zout = jnp.zeros((rpt, d), jnp.float32)

    mesh = plsc.VectorSubcoreMesh(core_axis_name="c", subcore_axis_name="s",
                                  num_cores=NC, num_subcores=NS)

    @functools.partial(
        pl.kernel,
        out_type=(jax.ShapeDtypeStruct((NC, npad, d), jnp.float32),
                  jax.ShapeDtypeStruct((NC, npad, L), jnp.float32)),
        mesh=mesh,
        compiler_params=pltpu.CompilerParams(use_tc_tiling_on_sc=False),
        scratch_types=[
            pltpu.VMEM((3, EB), jnp.int32),
            pltpu.VMEM((3, EB), jnp.int32),
            pltpu.VMEM((3, EB, L), jnp.float32),
            pltpu.VMEM((3, EB, L), jnp.float32),
            pltpu.VMEM((3, EB, L), jnp.float32),
            pltpu.VMEM((3, EB, d), jnp.float32),
            pltpu.VMEM_SHARED((npad, d), jnp.float32),
            pltpu.VMEM_SHARED((npad, L), jnp.float32),
            pltpu.SemaphoreType.DMA,
            pltpu.SemaphoreType.DMA,
            pltpu.SemaphoreType.DMA,
            pltpu.SemaphoreType.DMA,
            pltpu.SemaphoreType.DMA,
            pltpu.SemaphoreType.DMA,
        ],
    )
    def k(src_hbm, dst_hbm, asrc_hbm, adst_hbm, h_hbm, zden_hbm, zout_hbm,
          out_hbm, den_hbm,
          sidx, didx, srow, drow, wbuf, msg, out_acc, den_acc,
          g0, g1, g2, s0, s1, s2):
        gsem = (g0, g1, g2)
        ssem = (s0, s1, s2)
        c = lax.axis_index("c")
        s = lax.axis_index("s")
        r0 = s * rpt
        pltpu.sync_copy(zden_hbm, den_acc.at[pl.ds(r0, rpt)])
        pltpu.sync_copy(zout_hbm, out_acc.at[pl.ds(r0, rpt)])
        plsc.subcore_barrier()
        base = jnp.where(c == 0, s * ept0, NS * ept0 + s * ept1)
        nt_c = jnp.where(c == 0, nb0 // 3, nb1 // 3)
        lanemask = lax.iota(jnp.int32, L) < nheads

        def issue_gather(j, b):
            off = base + j * EB
            pltpu.sync_copy(src_hbm.at[pl.ds(off, EB)], sidx.at[b])
            pltpu.sync_copy(dst_hbm.at[pl.ds(off, EB)], didx.at[b])
            pltpu.async_copy(asrc_hbm.at[sidx.at[b]], srow.at[b], gsem[b])
            pltpu.async_copy(adst_hbm.at[didx.at[b]], drow.at[b], gsem[b])
            pltpu.async_copy(h_hbm.at[sidx.at[b]], msg.at[b], gsem[b])

        def wait_gather(b):
            pltpu.make_async_copy(asrc_hbm.at[sidx.at[b]], srow.at[b], gsem[b]).wait()
            pltpu.make_async_copy(adst_hbm.at[didx.at[b]], drow.at[b], gsem[b]).wait()
            pltpu.make_async_copy(h_hbm.at[sidx.at[b]], msg.at[b], gsem[b]).wait()

        def issue_scatter(b):
            pltpu.async_copy(wbuf.at[b], den_acc.at[didx.at[b]], ssem[b], add=True)
            pltpu.async_copy(msg.at[b], out_acc.at[didx.at[b]], ssem[b], add=True)

        def wait_scatter(b):
            pltpu.make_async_copy(wbuf.at[b], den_acc.at[didx.at[b]], ssem[b]).wait()
            pltpu.make_async_copy(msg.at[b], out_acc.at[didx.at[b]], ssem[b]).wait()

        def compute(b):
            def edge(e, carry):
                ev = srow[b, e] + drow[b, e]
                ev = jnp.maximum(ev, 0.2 * ev)
                wv = jnp.exp(ev)
                wv = jnp.where(lanemask, wv, 0.0)
                wbuf[b, e] = wv
                for v in range(d // L):
                    sc = wv[(v * L) // hid]
                    msg[b, e, pl.ds(v * L, L)] = msg[b, e, pl.ds(v * L, L)] * sc
                return carry
            lax.fori_loop(0, EB, edge, 0, unroll=2)

        def pipestep(j, k_, head=False, issue_next=True):
            wait_gather(k_)
            if not head:
                wait_scatter((k_ + 1) % 3)
            if issue_next:
                issue_gather(j + 1, (k_ + 1) % 3)
            compute(k_)
            issue_scatter(k_)

        issue_gather(0, 0)
        pipestep(0, 0, head=True)
        pipestep(1, 1, head=True)
        pipestep(2, 2)

        def triple(j3, carry):
            for k_ in range(3):
                pipestep(j3 * 3 + k_, k_)
            return carry

        lax.fori_loop(1, nt_c - 1, triple, 0)
        j0 = (nt_c - 1) * 3
        pipestep(j0, 0)
        pipestep(j0 + 1, 1)
        pipestep(j0 + 2, 2, issue_next=False)
        wait_scatter(1)
        wait_scatter(2)
        plsc.subcore_barrier()
        pltpu.sync_copy(out_acc.at[pl.ds(r0, rpt)],
                        out_hbm.at[c, pl.ds(r0, rpt)])
        pltpu.sync_copy(den_acc.at[pl.ds(r0, rpt)],
                        den_hbm.at[c, pl.ds(r0, rpt)])

    return k(src, dst, asrc, adst, h, zden, zout)


def _blk(n):
    for b in (1000, 500, 250, 200, 125, 100, 50, 40, 25, 20, 10, 8, 5, 4, 2, 1):
        if n % b == 0:
            return b
    return n


def _tc_pre(x, W, As, Ad):
    n, _ = x.shape
    dh = W.shape[1]
    blk = _blk(n)

    def body(x_ref, w_ref, a_ref, b_ref, h_ref, s_ref, t_ref):
        hv = jnp.dot(x_ref[...], w_ref[...], preferred_element_type=jnp.float32)
        h_ref[...] = hv
        s_ref[...] = jnp.dot(hv, a_ref[...], preferred_element_type=jnp.float32)
        t_ref[...] = jnp.dot(hv, b_ref[...], preferred_element_type=jnp.float32)

    return pl.pallas_call(
        body,
        grid=(n // blk,),
        in_specs=[pl.BlockSpec((blk, x.shape[1]), lambda i: (i, 0)),
                  pl.BlockSpec(W.shape, lambda i: (0, 0)),
                  pl.BlockSpec(As.shape, lambda i: (0, 0)),
                  pl.BlockSpec(Ad.shape, lambda i: (0, 0))],
        out_specs=[pl.BlockSpec((blk, dh), lambda i: (i, 0)),
                   pl.BlockSpec((blk, L), lambda i: (i, 0)),
                   pl.BlockSpec((blk, L), lambda i: (i, 0))],
        out_shape=[jax.ShapeDtypeStruct((n, dh), jnp.float32),
                   jax.ShapeDtypeStruct((n, L), jnp.float32),
                   jax.ShapeDtypeStruct((n, L), jnp.float32)],
    )(x, W, As, Ad)


def _tc_mid(p0, p1, dn0, dn1, R, b1, W2, As, Ad):
    n, d1 = p0.shape
    d2 = W2.shape[1]
    blk = _blk(n)

    def body(p0_ref, p1_ref, dn0_ref, dn1_ref, r_ref, b_ref, w_ref, a_ref,
             c_ref, h_ref, s_ref, t_ref):
        den = jnp.dot(dn0_ref[...] + dn1_ref[...], r_ref[...],
                      preferred_element_type=jnp.float32)
        hin = (p0_ref[...] + p1_ref[...]) / (den + 1e-16) + b_ref[...]
        hin = jnp.where(hin > 0, hin, jnp.exp(hin) - 1.0)
        h2 = jnp.dot(hin, w_ref[...], preferred_element_type=jnp.float32)
        h_ref[...] = h2
        s_ref[...] = jnp.dot(h2, a_ref[...], preferred_element_type=jnp.float32)
        t_ref[...] = jnp.dot(h2, c_ref[...], preferred_element_type=jnp.float32)

    return pl.pallas_call(
        body,
        grid=(n // blk,),
        in_specs=[pl.BlockSpec((blk, d1), lambda i: (i, 0)),
                  pl.BlockSpec((blk, d1), lambda i: (i, 0)),
                  pl.BlockSpec((blk, L), lambda i: (i, 0)),
                  pl.BlockSpec((blk, L), lambda i: (i, 0)),
                  pl.BlockSpec(R.shape, lambda i: (0, 0)),
                  pl.BlockSpec((1, d1), lambda i: (0, 0)),
                  pl.BlockSpec(W2.shape, lambda i: (0, 0)),
                  pl.BlockSpec(As.shape, lambda i: (0, 0)),
                  pl.BlockSpec(Ad.shape, lambda i: (0, 0))],
        out_specs=[pl.BlockSpec((blk, d2), lambda i: (i, 0)),
                   pl.BlockSpec((blk, L), lambda i: (i, 0)),
                   pl.BlockSpec((blk, L), lambda i: (i, 0))],
        out_shape=[jax.ShapeDtypeStruct((n, d2), jnp.float32),
                   jax.ShapeDtypeStruct((n, L), jnp.float32),
                   jax.ShapeDtypeStruct((n, L), jnp.float32)],
    )(p0, p1, dn0, dn1, R, b1, W2, As, Ad)


def _tc_fin(q0, q1, dn0, dn1, R, b2):
    n, d2 = q0.shape
    blk = _blk(n)

    def body(q0_ref, q1_ref, dn0_ref, dn1_ref, r_ref, b_ref, o_ref):
        den = jnp.dot(dn0_ref[...] + dn1_ref[...], r_ref[...],
                      preferred_element_type=jnp.float32)
        o_ref[...] = (q0_ref[...] + q1_ref[...]) / (den + 1e-16) + b_ref[...]

    return pl.pallas_call(
        body,
        grid=(n // blk,),
        in_specs=[pl.BlockSpec((blk, d2), lambda i: (i, 0)),
                  pl.BlockSpec((blk, d2), lambda i: (i, 0)),
                  pl.BlockSpec((blk, L), lambda i: (i, 0)),
                  pl.BlockSpec((blk, L), lambda i: (i, 0)),
                  pl.BlockSpec(R.shape, lambda i: (0, 0)),
                  pl.BlockSpec((1, d2), lambda i: (0, 0))],
        out_specs=pl.BlockSpec((blk, d2), lambda i: (i, 0)),
        out_shape=jax.ShapeDtypeStruct((n, d2), jnp.float32),
    )(q0, q1, dn0, dn1, R, b2)


def kernel(x, edge_index, W1, att_src1, att_dst1, b1, W2, att_src2, att_dst2, b2):
    n = x.shape[0]
    e = edge_index.shape[1]
    h1, hid1 = att_src1.shape
    d1 = h1 * hid1
    d2 = W2.shape[1]

    nbt = max(-(-e // (NW * EB)), 6)
    nbt = -(-nbt // 3) * 3
    ept = nbt * EB
    pad = ept * NW - e
    npad = ((n + 1 + NS - 1) // NS) * NS
    src = jnp.concatenate([edge_index[0], jnp.zeros((pad,), jnp.int32)])
    dst = jnp.concatenate(
        [edge_index[1], n + (jnp.arange(pad, dtype=jnp.int32) % (npad - n))])

    eye1 = jnp.eye(h1, L, dtype=jnp.float32)
    As1 = (att_src1[:, :, None] * eye1[:, None, :]).reshape(d1, L)
    Ad1 = (att_dst1[:, :, None] * eye1[:, None, :]).reshape(d1, L)
    eye2 = jnp.eye(1, L, dtype=jnp.float32)
    As2 = (att_src2[:, :, None] * eye2[:, None, :]).reshape(d2, L)
    Ad2 = (att_dst2[:, :, None] * eye2[:, None, :]).reshape(d2, L)
    R1 = jnp.repeat(jnp.eye(L, h1, dtype=jnp.float32), d1 // h1, axis=1)
    R2 = jnp.repeat(jnp.eye(L, 1, dtype=jnp.float32), d2, axis=1)

    tot = 2 * nbt

    def _split(frac0):
        return min(max(6, 3 * round(tot * frac0 / 3)), tot - 6)

    ha, s1, t1 = _tc_pre(x, W1, As1, Ad1)
    p, dn = _edge_pass(src, dst, s1, t1, ha, h1, _split(0.60))
    h2, s2, t2 = _tc_mid(p[0, :n], p[1, :n], dn[0, :n], dn[1, :n], R1,
                         b1.reshape(1, d1), W2, As2, Ad2)
    q, dn2 = _edge_pass(src, dst, s2, t2, h2, 1, _split(0.55))
    return _tc_fin(q[0, :n], q[1, :n], dn2[0, :n], dn2[1, :n], R2,
                   b2.reshape(1, d2))


# speedup vs baseline: 1.1701x; 1.0000x over previous
import functools
import jax
import jax.numpy as jnp
from jax import lax
from jax.experimental import pallas as pl
from jax.experimental.pallas import tpu as pltpu
from jax.experimental.pallas import tpu_sc as plsc

NC, NS, L = 2, 16, 16
NW = NC * NS
EB = 64


def _edge_pass(src, dst, asrc, adst, h, nheads, nb0):
    n, d = h.shape
    ept2 = 2 * (src.shape[0] // NW)
    nb1 = ept2 // EB - nb0
    ept0, ept1 = nb0 * EB, nb1 * EB
    npad = ((n + 1 + NS - 1) // NS) * NS
    rpt = npad // NS
    hid = d // nheads
    zden = jnp.zeros((rpt, L), jnp.float32)
    zout = jnp.zeros((rpt, d), jnp.float32)

    mesh = plsc.VectorSubcoreMesh(core_axis_name="c", subcore_axis_name="s",
                                  num_cores=NC, num_subcores=NS)

    @functools.partial(
        pl.kernel,
        out_type=(jax.ShapeDtypeStruct((NC, npad, d), jnp.float32),
                  jax.ShapeDtypeStruct((NC, npad, L), jnp.float32)),
        mesh=mesh,
        compiler_params=pltpu.CompilerParams(use_tc_tiling_on_sc=False),
        scratch_types=[
            pltpu.VMEM((3, EB), jnp.int32),
            pltpu.VMEM((3, EB), jnp.int32),
            pltpu.VMEM((3, EB, L), jnp.float32),
            pltpu.VMEM((3, EB, L), jnp.float32),
            pltpu.VMEM((3, EB, L), jnp.float32),
            pltpu.VMEM((3, EB, d), jnp.float32),
            pltpu.VMEM_SHARED((npad, d), jnp.float32),
            pltpu.VMEM_SHARED((npad, L), jnp.float32),
            pltpu.SemaphoreType.DMA,
            pltpu.SemaphoreType.DMA,
            pltpu.SemaphoreType.DMA,
            pltpu.SemaphoreType.DMA,
            pltpu.SemaphoreType.DMA,
            pltpu.SemaphoreType.DMA,
        ],
    )
    def k(src_hbm, dst_hbm, asrc_hbm, adst_hbm, h_hbm, zden_hbm, zout_hbm,
          out_hbm, den_hbm,
          sidx, didx, srow, drow, wbuf, msg, out_acc, den_acc,
          g0, g1, g2, s0, s1, s2):
        gsem = (g0, g1, g2)
        ssem = (s0, s1, s2)
        c = lax.axis_index("c")
        s = lax.axis_index("s")
        r0 = s * rpt
        pltpu.sync_copy(zden_hbm, den_acc.at[pl.ds(r0, rpt)])
        pltpu.sync_copy(zout_hbm, out_acc.at[pl.ds(r0, rpt)])
        plsc.subcore_barrier()
        base = jnp.where(c == 0, s * ept0, NS * ept0 + s * ept1)
        nt_c = jnp.where(c == 0, nb0 // 3, nb1 // 3)
        lanemask = lax.iota(jnp.int32, L) < nheads

        def issue_gather(j, b):
            off = base + j * EB
            pltpu.sync_copy(src_hbm.at[pl.ds(off, EB)], sidx.at[b])
            pltpu.sync_copy(dst_hbm.at[pl.ds(off, EB)], didx.at[b])
            pltpu.async_copy(asrc_hbm.at[sidx.at[b]], srow.at[b], gsem[b])
            pltpu.async_copy(adst_hbm.at[didx.at[b]], drow.at[b], gsem[b])
            pltpu.async_copy(h_hbm.at[sidx.at[b]], msg.at[b], gsem[b])

        def wait_gather(b):
            pltpu.make_async_copy(asrc_hbm.at[sidx.at[b]], srow.at[b], gsem[b]).wait()
            pltpu.make_async_copy(adst_hbm.at[didx.at[b]], drow.at[b], gsem[b]).wait()
            pltpu.make_async_copy(h_hbm.at[sidx.at[b]], msg.at[b], gsem[b]).wait()

        def issue_scatter(b):
            pltpu.async_copy(wbuf.at[b], den_acc.at[didx.at[b]], ssem[b], add=True)
            pltpu.async_copy(msg.at[b], out_acc.at[didx.at[b]], ssem[b], add=True)

        def wait_scatter(b):
            pltpu.make_async_copy(wbuf.at[b], den_acc.at[didx.at[b]], ssem[b]).wait()
            pltpu.make_async_copy(msg.at[b], out_acc.at[didx.at[b]], ssem[b]).wait()

        def compute(b):
            def edge(e, carry):
                ev = srow[b, e] + drow[b, e]
                ev = jnp.maximum(ev, 0.2 * ev)
                wv = jnp.exp(ev)
                wv = jnp.where(lanemask, wv, 0.0)
                wbuf[b, e] = wv
                for v in range(d // L):
                    sc = wv[(v * L) // hid]
                    msg[b, e, pl.ds(v * L, L)] = msg[b, e, pl.ds(v * L, L)] * sc
                return carry
            lax.fori_loop(0, EB, edge, 0, unroll=4)

        def pipestep(j, k_, head=False, issue_next=True):
            wait_gather(k_)
            if not head:
                wait_scatter((k_ + 1) % 3)
            if issue_next:
                issue_gather(j + 1, (k_ + 1) % 3)
            compute(k_)
            issue_scatter(k_)

        issue_gather(0, 0)
        pipestep(0, 0, head=True)
        pipestep(1, 1, head=True)
        pipestep(2, 2)

        def triple(j3, carry):
            for k_ in range(3):
                pipestep(j3 * 3 + k_, k_)
            return carry

        lax.fori_loop(1, nt_c - 1, triple, 0)
        j0 = (nt_c - 1) * 3
        pipestep(j0, 0)
        pipestep(j0 + 1, 1)
        pipestep(j0 + 2, 2, issue_next=False)
        wait_scatter(1)
        wait_scatter(2)
        plsc.subcore_barrier()
        pltpu.sync_copy(out_acc.at[pl.ds(r0, rpt)],
                        out_hbm.at[c, pl.ds(r0, rpt)])
        pltpu.sync_copy(den_acc.at[pl.ds(r0, rpt)],
                        den_hbm.at[c, pl.ds(r0, rpt)])

    return k(src, dst, asrc, adst, h, zden, zout)


def _blk(n):
    for b in (1000, 500, 250, 200, 125, 100, 50, 40, 25, 20, 10, 8, 5, 4, 2, 1):
        if n % b == 0:
            return b
    return n


def _tc_pre(x, W, As, Ad):
    n, _ = x.shape
    dh = W.shape[1]
    blk = _blk(n)

    def body(x_ref, w_ref, a_ref, b_ref, h_ref, s_ref, t_ref):
        hv = jnp.dot(x_ref[...], w_ref[...], preferred_element_type=jnp.float32)
        h_ref[...] = hv
        s_ref[...] = jnp.dot(hv, a_ref[...], preferred_element_type=jnp.float32)
        t_ref[...] = jnp.dot(hv, b_ref[...], preferred_element_type=jnp.float32)

    return pl.pallas_call(
        body,
        grid=(n // blk,),
        in_specs=[pl.BlockSpec((blk, x.shape[1]), lambda i: (i, 0)),
                  pl.BlockSpec(W.shape, lambda i: (0, 0)),
                  pl.BlockSpec(As.shape, lambda i: (0, 0)),
                  pl.BlockSpec(Ad.shape, lambda i: (0, 0))],
        out_specs=[pl.BlockSpec((blk, dh), lambda i: (i, 0)),
                   pl.BlockSpec((blk, L), lambda i: (i, 0)),
                   pl.BlockSpec((blk, L), lambda i: (i, 0))],
        out_shape=[jax.ShapeDtypeStruct((n, dh), jnp.float32),
                   jax.ShapeDtypeStruct((n, L), jnp.float32),
                   jax.ShapeDtypeStruct((n, L), jnp.float32)],
    )(x, W, As, Ad)


def _tc_mid(p0, p1, dn0, dn1, R, b1, W2, As, Ad):
    n, d1 = p0.shape
    d2 = W2.shape[1]
    blk = _blk(n)

    def body(p0_ref, p1_ref, dn0_ref, dn1_ref, r_ref, b_ref, w_ref, a_ref,
             c_ref, h_ref, s_ref, t_ref):
        den = jnp.dot(dn0_ref[...] + dn1_ref[...], r_ref[...],
                      preferred_element_type=jnp.float32)
        hin = (p0_ref[...] + p1_ref[...]) / (den + 1e-16) + b_ref[...]
        hin = jnp.where(hin > 0, hin, jnp.exp(hin) - 1.0)
        h2 = jnp.dot(hin, w_ref[...], preferred_element_type=jnp.float32)
        h_ref[...] = h2
        s_ref[...] = jnp.dot(h2, a_ref[...], preferred_element_type=jnp.float32)
        t_ref[...] = jnp.dot(h2, c_ref[...], preferred_element_type=jnp.float32)

    return pl.pallas_call(
        body,
        grid=(n // blk,),
        in_specs=[pl.BlockSpec((blk, d1), lambda i: (i, 0)),
                  pl.BlockSpec((blk, d1), lambda i: (i, 0)),
                  pl.BlockSpec((blk, L), lambda i: (i, 0)),
                  pl.BlockSpec((blk, L), lambda i: (i, 0)),
                  pl.BlockSpec(R.shape, lambda i: (0, 0)),
                  pl.BlockSpec((1, d1), lambda i: (0, 0)),
                  pl.BlockSpec(W2.shape, lambda i: (0, 0)),
                  pl.BlockSpec(As.shape, lambda i: (0, 0)),
                  pl.BlockSpec(Ad.shape, lambda i: (0, 0))],
        out_specs=[pl.BlockSpec((blk, d2), lambda i: (i, 0)),
                   pl.BlockSpec((blk, L), lambda i: (i, 0)),
                   pl.BlockSpec((blk, L), lambda i: (i, 0))],
        out_shape=[jax.ShapeDtypeStruct((n, d2), jnp.float32),
                   jax.ShapeDtypeStruct((n, L), jnp.float32),
                   jax.ShapeDtypeStruct((n, L), jnp.float32)],
    )(p0, p1, dn0, dn1, R, b1, W2, As, Ad)


def _tc_fin(q0, q1, dn0, dn1, R, b2):
    n, d2 = q0.shape
    blk = _blk(n)

    def body(q0_ref, q1_ref, dn0_ref, dn1_ref, r_ref, b_ref, o_ref):
        den = jnp.dot(dn0_ref[...] + dn1_ref[...], r_ref[...],
                      preferred_element_type=jnp.float32)
        o_ref[...] = (q0_ref[...] + q1_ref[...]) / (den + 1e-16) + b_ref[...]

    return pl.pallas_call(
        body,
        grid=(n // blk,),
        in_specs=[pl.BlockSpec((blk, d2), lambda i: (i, 0)),
                  pl.BlockSpec((blk, d2), lambda i: (i, 0)),
                  pl.BlockSpec((blk, L), lambda i: (i, 0)),
                  pl.BlockSpec((blk, L), lambda i: (i, 0)),
                  pl.BlockSpec(R.shape, lambda i: (0, 0)),
                  pl.BlockSpec((1, d2), lambda i: (0, 0))],
        out_specs=pl.BlockSpec((blk, d2), lambda i: (i, 0)),
        out_shape=jax.ShapeDtypeStruct((n, d2), jnp.float32),
    )(q0, q1, dn0, dn1, R, b2)


def kernel(x, edge_index, W1, att_src1, att_dst1, b1, W2, att_src2, att_dst2, b2):
    n = x.shape[0]
    e = edge_index.shape[1]
    h1, hid1 = att_src1.shape
    d1 = h1 * hid1
    d2 = W2.shape[1]

    nbt = max(-(-e // (NW * EB)), 6)
    nbt = -(-nbt // 3) * 3
    ept = nbt * EB
    pad = ept * NW - e
    npad = ((n + 1 + NS - 1) // NS) * NS
    src = jnp.concatenate([edge_index[0], jnp.zeros((pad,), jnp.int32)])
    dst = jnp.concatenate(
        [edge_index[1], n + (jnp.arange(pad, dtype=jnp.int32) % (npad - n))])

    eye1 = jnp.eye(h1, L, dtype=jnp.float32)
    As1 = (att_src1[:, :, None] * eye1[:, None, :]).reshape(d1, L)
    Ad1 = (att_dst1[:, :, None] * eye1[:, None, :]).reshape(d1, L)
    eye2 = jnp.eye(1, L, dtype=jnp.float32)
    As2 = (att_src2[:, :, None] * eye2[:, None, :]).reshape(d2, L)
    Ad2 = (att_dst2[:, :, None] * eye2[:, None, :]).reshape(d2, L)
    R1 = jnp.repeat(jnp.eye(L, h1, dtype=jnp.float32), d1 // h1, axis=1)
    R2 = jnp.repeat(jnp.eye(L, 1, dtype=jnp.float32), d2, axis=1)

    tot = 2 * nbt

    def _split(frac0):
        return min(max(6, 3 * round(tot * frac0 / 3)), tot - 6)

    ha, s1, t1 = _tc_pre(x, W1, As1, Ad1)
    p, dn = _edge_pass(src, dst, s1, t1, ha, h1, _split(0.60))
    h2, s2, t2 = _tc_mid(p[0, :n], p[1, :n], dn[0, :n], dn[1, :n], R1,
                         b1.reshape(1, d1), W2, As2, Ad2)
    q, dn2 = _edge_pass(src, dst, s2, t2, h2, 1, _split(0.55))
    return _tc_fin(q[0, :n], q[1, :n], dn2[0, :n], dn2[1, :n], R2,
                   b2.reshape(1, d2))


# speedup vs baseline: 1.2184x; 1.0413x over previous
import functools
import jax
import jax.numpy as jnp
from jax import lax
from jax.experimental import pallas as pl
from jax.experimental.pallas import tpu as pltpu
from jax.experimental.pallas import tpu_sc as plsc

NC, NS, L = 2, 16, 16
NW = NC * NS
EB = 64


def _edge_pass(src, dst, haug, adst, nheads, nb0):
    n, wd = haug.shape
    d = wd - L
    ept2 = 2 * (src.shape[0] // NW)
    nb1 = ept2 // EB - nb0
    ept0, ept1 = nb0 * EB, nb1 * EB
    npad = ((n + 1 + NS - 1) // NS) * NS
    rpt = npad // NS
    hid = d // nheads
    zacc = jnp.zeros((rpt, wd), jnp.float32)

    mesh = plsc.VectorSubcoreMesh(core_axis_name="c", subcore_axis_name="s",
                                  num_cores=NC, num_subcores=NS)

    @functools.partial(
        pl.kernel,
        out_type=jax.ShapeDtypeStruct((NC, npad, wd), jnp.float32),
        mesh=mesh,
        compiler_params=pltpu.CompilerParams(use_tc_tiling_on_sc=False),
        scratch_types=[
            pltpu.VMEM((3, EB), jnp.int32),
            pltpu.VMEM((3, EB), jnp.int32),
            pltpu.VMEM((3, EB, L), jnp.float32),
            pltpu.VMEM((3, EB, wd), jnp.float32),
            pltpu.VMEM_SHARED((npad, wd), jnp.float32),
            pltpu.SemaphoreType.DMA,
            pltpu.SemaphoreType.DMA,
            pltpu.SemaphoreType.DMA,
            pltpu.SemaphoreType.DMA,
            pltpu.SemaphoreType.DMA,
            pltpu.SemaphoreType.DMA,
        ],
    )
    def k(src_hbm, dst_hbm, haug_hbm, adst_hbm, zacc_hbm, out_hbm,
          sidx, didx, drow, mbuf, acc, g0, g1, g2, s0, s1, s2):
        gsem = (g0, g1, g2)
        ssem = (s0, s1, s2)
        c = lax.axis_index("c")
        s = lax.axis_index("s")
        r0 = s * rpt
        pltpu.sync_copy(zacc_hbm, acc.at[pl.ds(r0, rpt)])
        plsc.subcore_barrier()
        base = jnp.where(c == 0, s * ept0, NS * ept0 + s * ept1)
        nt_c = jnp.where(c == 0, nb0 // 3, nb1 // 3)
        lanemask = lax.iota(jnp.int32, L) < nheads

        def issue_gather(j, b):
            off = base + j * EB
            pltpu.sync_copy(src_hbm.at[pl.ds(off, EB)], sidx.at[b])
            pltpu.sync_copy(dst_hbm.at[pl.ds(off, EB)], didx.at[b])
            pltpu.async_copy(haug_hbm.at[sidx.at[b]], mbuf.at[b], gsem[b])
            pltpu.async_copy(adst_hbm.at[didx.at[b]], drow.at[b], gsem[b])

        def wait_gather(b):
            pltpu.make_async_copy(haug_hbm.at[sidx.at[b]], mbuf.at[b],
                                  gsem[b]).wait()
            pltpu.make_async_copy(adst_hbm.at[didx.at[b]], drow.at[b],
                                  gsem[b]).wait()

        def issue_scatter(b):
            pltpu.async_copy(mbuf.at[b], acc.at[didx.at[b]], ssem[b],
                             add=True)

        def wait_scatter(b):
            pltpu.make_async_copy(mbuf.at[b], acc.at[didx.at[b]],
                                  ssem[b]).wait()

        def compute(b):
            def edge(e, carry):
                ev = mbuf[b, e, pl.ds(d, L)] + drow[b, e]
                ev = jnp.maximum(ev, 0.2 * ev)
                wv = jnp.exp(ev)
                wv = jnp.where(lanemask, wv, 0.0)
                mbuf[b, e, pl.ds(d, L)] = wv
                for v in range(d // L):
                    sc = wv[(v * L) // hid]
                    mbuf[b, e, pl.ds(v * L, L)] = mbuf[b, e, pl.ds(v * L, L)] * sc
                return carry
            lax.fori_loop(0, EB, edge, 0, unroll=2)

        def pipestep(j, k_, head=False, issue_next=True):
            wait_gather(k_)
            if not head:
                wait_scatter((k_ + 1) % 3)
            if issue_next:
                issue_gather(j + 1, (k_ + 1) % 3)
            compute(k_)
            issue_scatter(k_)

        issue_gather(0, 0)
        pipestep(0, 0, head=True)
        pipestep(1, 1, head=True)
        pipestep(2, 2)

        def triple(j3, carry):
            for k_ in range(3):
                pipestep(j3 * 3 + k_, k_)
            return carry

        lax.fori_loop(1, nt_c - 1, triple, 0)
        j0 = (nt_c - 1) * 3
        pipestep(j0, 0)
        pipestep(j0 + 1, 1)
        pipestep(j0 + 2, 2, issue_next=False)
        wait_scatter(1)
        wait_scatter(2)
        plsc.subcore_barrier()
        pltpu.sync_copy(acc.at[pl.ds(r0, rpt)], out_hbm.at[c, pl.ds(r0, rpt)])

    return k(src, dst, haug, adst, zacc)


def _blk(n):
    for b in (1000, 500, 250, 200, 125, 100, 50, 40, 25, 20, 10, 8, 5, 4, 2, 1):
        if n % b == 0:
            return b
    return n


def _tc_pre(x, W, As, Ad):
    n = x.shape[0]
    dh = W.shape[1]

    blk = _blk(n)

    def body(x_ref, w_ref, a_ref, b_ref, o_ref, t_ref):
        hv = jnp.dot(x_ref[...], w_ref[...], preferred_element_type=jnp.float32)
        o_ref[:, :dh] = hv
        o_ref[:, dh:] = jnp.dot(hv, a_ref[...], preferred_element_type=jnp.float32)
        t_ref[...] = jnp.dot(hv, b_ref[...], preferred_element_type=jnp.float32)

    return pl.pallas_call(
        body,
        grid=(n // blk,),
        in_specs=[pl.BlockSpec((blk, x.shape[1]), lambda i: (i, 0)),
                  pl.BlockSpec(W.shape, lambda i: (0, 0)),
                  pl.BlockSpec(As.shape, lambda i: (0, 0)),
                  pl.BlockSpec(Ad.shape, lambda i: (0, 0))],
        out_specs=[pl.BlockSpec((blk, dh + L), lambda i: (i, 0)),
                   pl.BlockSpec((blk, L), lambda i: (i, 0))],
        out_shape=[jax.ShapeDtypeStruct((n, dh + L), jnp.float32),
                   jax.ShapeDtypeStruct((n, L), jnp.float32)],
    )(x, W, As, Ad)


def _tc_mid(p0, p1, R, b1, W2, As, Ad):
    n, wd = p0.shape
    d1 = wd - L
    d2 = W2.shape[1]
    blk = _blk(n)

    def body(p0_ref, p1_ref, r_ref, b_ref, w_ref, a_ref, c_ref, o_ref, t_ref):
        sv = p0_ref[...] + p1_ref[...]
        den = jnp.dot(sv[:, d1:], r_ref[...], preferred_element_type=jnp.float32)
        hin = sv[:, :d1] / (den + 1e-16) + b_ref[...]
        hin = jnp.where(hin > 0, hin, jnp.exp(hin) - 1.0)
        h2 = jnp.dot(hin, w_ref[...], preferred_element_type=jnp.float32)
        o_ref[:, :d2] = h2
        o_ref[:, d2:] = jnp.dot(h2, a_ref[...], preferred_element_type=jnp.float32)
        t_ref[...] = jnp.dot(h2, c_ref[...], preferred_element_type=jnp.float32)

    return pl.pallas_call(
        body,
        grid=(n // blk,),
        in_specs=[pl.BlockSpec((blk, wd), lambda i: (i, 0)),
                  pl.BlockSpec((blk, wd), lambda i: (i, 0)),
                  pl.BlockSpec(R.shape, lambda i: (0, 0)),
                  pl.BlockSpec((1, d1), lambda i: (0, 0)),
                  pl.BlockSpec(W2.shape, lambda i: (0, 0)),
                  pl.BlockSpec(As.shape, lambda i: (0, 0)),
                  pl.BlockSpec(Ad.shape, lambda i: (0, 0))],
        out_specs=[pl.BlockSpec((blk, d2 + L), lambda i: (i, 0)),
                   pl.BlockSpec((blk, L), lambda i: (i, 0))],
        out_shape=[jax.ShapeDtypeStruct((n, d2 + L), jnp.float32),
                   jax.ShapeDtypeStruct((n, L), jnp.float32)],
    )(p0, p1, R, b1, W2, As, Ad)


def _tc_fin(q0, q1, R, b2):
    n, wd = q0.shape
    d2 = wd - L
    blk = _blk(n)

    def body(q0_ref, q1_ref, r_ref, b_ref, o_ref):
        sv = q0_ref[...] + q1_ref[...]
        den = jnp.dot(sv[:, d2:], r_ref[...], preferred_element_type=jnp.float32)
        o_ref[...] = sv[:, :d2] / (den + 1e-16) + b_ref[...]

    return pl.pallas_call(
        body,
        grid=(n // blk,),
        in_specs=[pl.BlockSpec((blk, wd), lambda i: (i, 0)),
                  pl.BlockSpec((blk, wd), lambda i: (i, 0)),
                  pl.BlockSpec(R.shape, lambda i: (0, 0)),
                  pl.BlockSpec((1, d2), lambda i: (0, 0))],
        out_specs=pl.BlockSpec((blk, d2), lambda i: (i, 0)),
        out_shape=jax.ShapeDtypeStruct((n, d2), jnp.float32),
    )(q0, q1, R, b2)


def kernel(x, edge_index, W1, att_src1, att_dst1, b1, W2, att_src2, att_dst2, b2):
    n = x.shape[0]
    e = edge_index.shape[1]
    h1, hid1 = att_src1.shape
    d1 = h1 * hid1
    d2 = W2.shape[1]

    nbt = max(-(-e // (NW * EB)), 6)
    nbt = -(-nbt // 3) * 3
    ept = nbt * EB
    pad = ept * NW - e
    npad = ((n + 1 + NS - 1) // NS) * NS
    src = jnp.concatenate([edge_index[0], jnp.zeros((pad,), jnp.int32)])
    dst = jnp.concatenate(
        [edge_index[1], n + (jnp.arange(pad, dtype=jnp.int32) % (npad - n))])

    eye1 = jnp.eye(h1, L, dtype=jnp.float32)
    As1 = (att_src1[:, :, None] * eye1[:, None, :]).reshape(d1, L)
    Ad1 = (att_dst1[:, :, None] * eye1[:, None, :]).reshape(d1, L)
    eye2 = jnp.eye(1, L, dtype=jnp.float32)
    As2 = (att_src2[:, :, None] * eye2[:, None, :]).reshape(d2, L)
    Ad2 = (att_dst2[:, :, None] * eye2[:, None, :]).reshape(d2, L)
    R1 = jnp.repeat(jnp.eye(L, h1, dtype=jnp.float32), d1 // h1, axis=1)
    R2 = jnp.repeat(jnp.eye(L, 1, dtype=jnp.float32), d2, axis=1)

    tot = 2 * nbt

    def _split(frac0):
        return min(max(6, 3 * round(tot * frac0 / 3)), tot - 6)

    ha, t1 = _tc_pre(x, W1, As1, Ad1)
    p = _edge_pass(src, dst, ha, t1, h1, _split(0.60))
    h2a, t2 = _tc_mid(p[0, :n], p[1, :n], R1, b1.reshape(1, d1), W2, As2, Ad2)
    q = _edge_pass(src, dst, h2a, t2, 1, _split(0.55))
    return _tc_fin(q[0, :n], q[1, :n], R2, b2.reshape(1, d2))


# speedup vs baseline: 1.2704x; 1.0426x over previous
import functools
import jax
import jax.numpy as jnp
from jax import lax
from jax.experimental import pallas as pl
from jax.experimental.pallas import tpu as pltpu
from jax.experimental.pallas import tpu_sc as plsc

NC, NS, L = 2, 16, 16
NW = NC * NS
EB = 64


def _edge_pass(src, dst, haug, adst, nheads, nb0):
    n, wd = haug.shape
    d = wd - L
    ept2 = 2 * (src.shape[0] // NW)
    nb1 = ept2 // EB - nb0
    ept0, ept1 = nb0 * EB, nb1 * EB
    npad = ((n + 1 + NS - 1) // NS) * NS
    rpt = npad // NS
    hid = d // nheads
    zacc = jnp.zeros((rpt, wd), jnp.float32)

    mesh = plsc.VectorSubcoreMesh(core_axis_name="c", subcore_axis_name="s",
                                  num_cores=NC, num_subcores=NS)

    @functools.partial(
        pl.kernel,
        out_type=jax.ShapeDtypeStruct((NC, npad, wd), jnp.float32),
        mesh=mesh,
        compiler_params=pltpu.CompilerParams(use_tc_tiling_on_sc=False),
        scratch_types=[
            pltpu.VMEM((3, EB), jnp.int32),
            pltpu.VMEM((3, EB), jnp.int32),
            pltpu.VMEM((3, EB, L), jnp.float32),
            pltpu.VMEM((3, EB, wd), jnp.float32),
            pltpu.VMEM_SHARED((npad, wd), jnp.float32),
            pltpu.SemaphoreType.DMA,
            pltpu.SemaphoreType.DMA,
            pltpu.SemaphoreType.DMA,
            pltpu.SemaphoreType.DMA,
            pltpu.SemaphoreType.DMA,
            pltpu.SemaphoreType.DMA,
        ],
    )
    def k(src_hbm, dst_hbm, haug_hbm, adst_hbm, zacc_hbm, out_hbm,
          sidx, didx, drow, mbuf, acc, g0, g1, g2, s0, s1, s2):
        gsem = (g0, g1, g2)
        ssem = (s0, s1, s2)
        c = lax.axis_index("c")
        s = lax.axis_index("s")
        r0 = s * rpt
        pltpu.sync_copy(zacc_hbm, acc.at[pl.ds(r0, rpt)])
        plsc.subcore_barrier()
        base = jnp.where(c == 0, s * ept0, NS * ept0 + s * ept1)
        nt_c = jnp.where(c == 0, nb0 // 3, nb1 // 3)
        lanemask = lax.iota(jnp.int32, L) < nheads

        def issue_gather(j, b):
            off = base + j * EB
            pltpu.sync_copy(src_hbm.at[pl.ds(off, EB)], sidx.at[b])
            pltpu.sync_copy(dst_hbm.at[pl.ds(off, EB)], didx.at[b])
            pltpu.async_copy(haug_hbm.at[sidx.at[b]], mbuf.at[b], gsem[b])
            pltpu.async_copy(adst_hbm.at[didx.at[b]], drow.at[b], gsem[b])

        def wait_gather(b):
            pltpu.make_async_copy(haug_hbm.at[sidx.at[b]], mbuf.at[b],
                                  gsem[b]).wait()
            pltpu.make_async_copy(adst_hbm.at[didx.at[b]], drow.at[b],
                                  gsem[b]).wait()

        def issue_scatter(b):
            pltpu.async_copy(mbuf.at[b], acc.at[didx.at[b]], ssem[b],
                             add=True)

        def wait_scatter(b):
            pltpu.make_async_copy(mbuf.at[b], acc.at[didx.at[b]],
                                  ssem[b]).wait()

        def compute(b):
            def edge(e, carry):
                ev = mbuf[b, e, pl.ds(d, L)] + drow[b, e]
                ev = jnp.maximum(ev, 0.2 * ev)
                wv = jnp.exp(ev)
                wv = jnp.where(lanemask, wv, 0.0)
                mbuf[b, e, pl.ds(d, L)] = wv
                for v in range(d // L):
                    sc = wv[(v * L) // hid]
                    mbuf[b, e, pl.ds(v * L, L)] = mbuf[b, e, pl.ds(v * L, L)] * sc
                return carry
            lax.fori_loop(0, EB, edge, 0, unroll=2)

        def pipestep(j, k_, head=False, issue_next=True):
            wait_gather(k_)
            if not head:
                wait_scatter((k_ + 1) % 3)
            if issue_next:
                issue_gather(j + 1, (k_ + 1) % 3)
            compute(k_)
            issue_scatter(k_)

        issue_gather(0, 0)
        pipestep(0, 0, head=True)
        pipestep(1, 1, head=True)
        pipestep(2, 2)

        def triple(j3, carry):
            for k_ in range(3):
                pipestep(j3 * 3 + k_, k_)
            return carry

        lax.fori_loop(1, nt_c - 1, triple, 0)
        j0 = (nt_c - 1) * 3
        pipestep(j0, 0)
        pipestep(j0 + 1, 1)
        pipestep(j0 + 2, 2, issue_next=False)
        wait_scatter(1)
        wait_scatter(2)
        plsc.subcore_barrier()
        pltpu.sync_copy(acc.at[pl.ds(r0, rpt)], out_hbm.at[c, pl.ds(r0, rpt)])

    return k(src, dst, haug, adst, zacc)


def _blk(n):
    for b in (1000, 500, 250, 200, 125, 100, 50, 40, 25, 20, 10, 8, 5, 4, 2, 1):
        if n % b == 0:
            return b
    return n


def _tc_pre(x, W, As, Ad):
    n = x.shape[0]
    dh = W.shape[1]

    blk = _blk(n)

    def body(x_ref, w_ref, a_ref, b_ref, o_ref, t_ref):
        hv = jnp.dot(x_ref[...], w_ref[...], preferred_element_type=jnp.float32)
        o_ref[:, :dh] = hv
        o_ref[:, dh:] = jnp.dot(hv, a_ref[...], preferred_element_type=jnp.float32)
        t_ref[...] = jnp.dot(hv, b_ref[...], preferred_element_type=jnp.float32)

    return pl.pallas_call(
        body,
        grid=(n // blk,),
        in_specs=[pl.BlockSpec((blk, x.shape[1]), lambda i: (i, 0)),
                  pl.BlockSpec(W.shape, lambda i: (0, 0)),
                  pl.BlockSpec(As.shape, lambda i: (0, 0)),
                  pl.BlockSpec(Ad.shape, lambda i: (0, 0))],
        out_specs=[pl.BlockSpec((blk, dh + L), lambda i: (i, 0)),
                   pl.BlockSpec((blk, L), lambda i: (i, 0))],
        out_shape=[jax.ShapeDtypeStruct((n, dh + L), jnp.float32),
                   jax.ShapeDtypeStruct((n, L), jnp.float32)],
    )(x, W, As, Ad)


def _tc_mid(p0, p1, R, b1, W2, As, Ad):
    n, wd = p0.shape
    d1 = wd - L
    d2 = W2.shape[1]
    blk = _blk(n)

    def body(p0_ref, p1_ref, r_ref, b_ref, w_ref, a_ref, c_ref, o_ref, t_ref):
        sv = p0_ref[...] + p1_ref[...]
        den = jnp.dot(sv[:, d1:], r_ref[...], preferred_element_type=jnp.float32)
        hin = sv[:, :d1] / (den + 1e-16) + b_ref[...]
        hin = jnp.where(hin > 0, hin, jnp.exp(hin) - 1.0)
        h2 = jnp.dot(hin, w_ref[...], preferred_element_type=jnp.float32)
        o_ref[:, :d2] = h2
        o_ref[:, d2:] = jnp.dot(h2, a_ref[...], preferred_element_type=jnp.float32)
        t_ref[...] = jnp.dot(h2, c_ref[...], preferred_element_type=jnp.float32)

    return pl.pallas_call(
        body,
        grid=(n // blk,),
        in_specs=[pl.BlockSpec((blk, wd), lambda i: (i, 0)),
                  pl.BlockSpec((blk, wd), lambda i: (i, 0)),
                  pl.BlockSpec(R.shape, lambda i: (0, 0)),
                  pl.BlockSpec((1, d1), lambda i: (0, 0)),
                  pl.BlockSpec(W2.shape, lambda i: (0, 0)),
                  pl.BlockSpec(As.shape, lambda i: (0, 0)),
                  pl.BlockSpec(Ad.shape, lambda i: (0, 0))],
        out_specs=[pl.BlockSpec((blk, d2 + L), lambda i: (i, 0)),
                   pl.BlockSpec((blk, L), lambda i: (i, 0))],
        out_shape=[jax.ShapeDtypeStruct((n, d2 + L), jnp.float32),
                   jax.ShapeDtypeStruct((n, L), jnp.float32)],
    )(p0, p1, R, b1, W2, As, Ad)


def _tc_fin(q0, q1, R, b2):
    n, wd = q0.shape
    d2 = wd - L
    blk = _blk(n)

    def body(q0_ref, q1_ref, r_ref, b_ref, o_ref):
        sv = q0_ref[...] + q1_ref[...]
        den = jnp.dot(sv[:, d2:], r_ref[...], preferred_element_type=jnp.float32)
        o_ref[...] = sv[:, :d2] / (den + 1e-16) + b_ref[...]

    return pl.pallas_call(
        body,
        grid=(n // blk,),
        in_specs=[pl.BlockSpec((blk, wd), lambda i: (i, 0)),
                  pl.BlockSpec((blk, wd), lambda i: (i, 0)),
                  pl.BlockSpec(R.shape, lambda i: (0, 0)),
                  pl.BlockSpec((1, d2), lambda i: (0, 0))],
        out_specs=pl.BlockSpec((blk, d2), lambda i: (i, 0)),
        out_shape=jax.ShapeDtypeStruct((n, d2), jnp.float32),
    )(q0, q1, R, b2)


def kernel(x, edge_index, W1, att_src1, att_dst1, b1, W2, att_src2, att_dst2, b2):
    n = x.shape[0]
    e = edge_index.shape[1]
    h1, hid1 = att_src1.shape
    d1 = h1 * hid1
    d2 = W2.shape[1]

    nbt = max(-(-e // (NW * EB)), 6)
    nbt = -(-nbt // 3) * 3
    ept = nbt * EB
    pad = ept * NW - e
    npad = ((n + 1 + NS - 1) // NS) * NS
    src = jnp.concatenate([edge_index[0], jnp.zeros((pad,), jnp.int32)])
    dst = jnp.concatenate(
        [edge_index[1], n + (jnp.arange(pad, dtype=jnp.int32) % (npad - n))])

    eye1 = jnp.eye(h1, L, dtype=jnp.float32)
    As1 = (att_src1[:, :, None] * eye1[:, None, :]).reshape(d1, L)
    Ad1 = (att_dst1[:, :, None] * eye1[:, None, :]).reshape(d1, L)
    eye2 = jnp.eye(1, L, dtype=jnp.float32)
    As2 = (att_src2[:, :, None] * eye2[:, None, :]).reshape(d2, L)
    Ad2 = (att_dst2[:, :, None] * eye2[:, None, :]).reshape(d2, L)
    R1 = jnp.repeat(jnp.eye(L, h1, dtype=jnp.float32), d1 // h1, axis=1)
    R2 = jnp.repeat(jnp.eye(L, 1, dtype=jnp.float32), d2, axis=1)

    tot = 2 * nbt

    def _split(frac0):
        return min(max(6, 3 * round(tot * frac0 / 3)), tot - 6)

    ha, t1 = _tc_pre(x, W1, As1, Ad1)
    p = _edge_pass(src, dst, ha, t1, h1, _split(0.60))
    h2a, t2 = _tc_mid(p[0, :n], p[1, :n], R1, b1.reshape(1, d1), W2, As2, Ad2)
    q = _edge_pass(src, dst, h2a, t2, 1, _split(0.62))
    return _tc_fin(q[0, :n], q[1, :n], R2, b2.reshape(1, d2))


# speedup vs baseline: 1.4570x; 1.1469x over previous
import functools
import jax
import jax.numpy as jnp
from jax import lax
from jax.experimental import pallas as pl
from jax.experimental.pallas import tpu as pltpu
from jax.experimental.pallas import tpu_sc as plsc

NC, NS, L = 2, 16, 16
NW = NC * NS
EB = 80


def _edge_pass(src, dst, haug, adst, nheads, nb0):
    n, wd = haug.shape
    d = wd - L
    ept2 = 2 * (src.shape[0] // NW)
    nb1 = ept2 // EB - nb0
    ept0, ept1 = nb0 * EB, nb1 * EB
    npad = ((n + 1 + NS - 1) // NS) * NS
    rpt = npad // NS
    hid = d // nheads
    zacc = jnp.zeros((rpt, wd), jnp.float32)

    mesh = plsc.VectorSubcoreMesh(core_axis_name="c", subcore_axis_name="s",
                                  num_cores=NC, num_subcores=NS)

    @functools.partial(
        pl.kernel,
        out_type=jax.ShapeDtypeStruct((NC, npad, wd), jnp.float32),
        mesh=mesh,
        compiler_params=pltpu.CompilerParams(use_tc_tiling_on_sc=False),
        scratch_types=[
            pltpu.VMEM((3, EB), jnp.int32),
            pltpu.VMEM((3, EB), jnp.int32),
            pltpu.VMEM((3, EB, L), jnp.float32),
            pltpu.VMEM((3, EB, wd), jnp.float32),
            pltpu.VMEM_SHARED((npad, wd), jnp.float32),
            pltpu.SemaphoreType.DMA,
            pltpu.SemaphoreType.DMA,
            pltpu.SemaphoreType.DMA,
            pltpu.SemaphoreType.DMA,
            pltpu.SemaphoreType.DMA,
            pltpu.SemaphoreType.DMA,
        ],
    )
    def k(src_hbm, dst_hbm, haug_hbm, adst_hbm, zacc_hbm, out_hbm,
          sidx, didx, drow, mbuf, acc, g0, g1, g2, s0, s1, s2):
        gsem = (g0, g1, g2)
        ssem = (s0, s1, s2)
        c = lax.axis_index("c")
        s = lax.axis_index("s")
        r0 = s * rpt
        pltpu.sync_copy(zacc_hbm, acc.at[pl.ds(r0, rpt)])
        plsc.subcore_barrier()
        base = jnp.where(c == 0, s * ept0, NS * ept0 + s * ept1)
        nt_c = jnp.where(c == 0, nb0 // 3, nb1 // 3)
        lanemask = lax.iota(jnp.int32, L) < nheads

        def issue_gather(j, b):
            off = base + j * EB
            pltpu.sync_copy(src_hbm.at[pl.ds(off, EB)], sidx.at[b])
            pltpu.sync_copy(dst_hbm.at[pl.ds(off, EB)], didx.at[b])
            pltpu.async_copy(haug_hbm.at[sidx.at[b]], mbuf.at[b], gsem[b])
            pltpu.async_copy(adst_hbm.at[didx.at[b]], drow.at[b], gsem[b])

        def wait_gather(b):
            pltpu.make_async_copy(haug_hbm.at[sidx.at[b]], mbuf.at[b],
                                  gsem[b]).wait()
            pltpu.make_async_copy(adst_hbm.at[didx.at[b]], drow.at[b],
                                  gsem[b]).wait()

        def issue_scatter(b):
            pltpu.async_copy(mbuf.at[b], acc.at[didx.at[b]], ssem[b],
                             add=True)

        def wait_scatter(b):
            pltpu.make_async_copy(mbuf.at[b], acc.at[didx.at[b]],
                                  ssem[b]).wait()

        def compute(b):
            def edge(e, carry):
                ev = mbuf[b, e, pl.ds(d, L)] + drow[b, e]
                ev = jnp.maximum(ev, 0.2 * ev)
                wv = jnp.exp(ev)
                wv = jnp.where(lanemask, wv, 0.0)
                mbuf[b, e, pl.ds(d, L)] = wv
                for v in range(d // L):
                    sc = wv[(v * L) // hid]
                    mbuf[b, e, pl.ds(v * L, L)] = mbuf[b, e, pl.ds(v * L, L)] * sc
                return carry
            lax.fori_loop(0, EB, edge, 0, unroll=2)

        def pipestep(j, k_, head=False, issue_next=True):
            wait_gather(k_)
            if not head:
                wait_scatter((k_ + 1) % 3)
            if issue_next:
                issue_gather(j + 1, (k_ + 1) % 3)
            compute(k_)
            issue_scatter(k_)

        issue_gather(0, 0)
        pipestep(0, 0, head=True)
        pipestep(1, 1, head=True)
        pipestep(2, 2)

        def triple(j3, carry):
            for k_ in range(3):
                pipestep(j3 * 3 + k_, k_)
            return carry

        lax.fori_loop(1, nt_c - 1, triple, 0)
        j0 = (nt_c - 1) * 3
        pipestep(j0, 0)
        pipestep(j0 + 1, 1)
        pipestep(j0 + 2, 2, issue_next=False)
        wait_scatter(1)
        wait_scatter(2)
        plsc.subcore_barrier()
        pltpu.sync_copy(acc.at[pl.ds(r0, rpt)], out_hbm.at[c, pl.ds(r0, rpt)])

    return k(src, dst, haug, adst, zacc)


def _blk(n):
    for b in (1000, 500, 250, 200, 125, 100, 50, 40, 25, 20, 10, 8, 5, 4, 2, 1):
        if n % b == 0:
            return b
    return n


def _tc_pre(x, W, As, Ad):
    n = x.shape[0]
    dh = W.shape[1]

    blk = _blk(n)

    def body(x_ref, w_ref, a_ref, b_ref, o_ref, t_ref):
        hv = jnp.dot(x_ref[...], w_ref[...], preferred_element_type=jnp.float32)
        o_ref[:, :dh] = hv
        o_ref[:, dh:] = jnp.dot(hv, a_ref[...], preferred_element_type=jnp.float32)
        t_ref[...] = jnp.dot(hv, b_ref[...], preferred_element_type=jnp.float32)

    return pl.pallas_call(
        body,
        grid=(n // blk,),
        in_specs=[pl.BlockSpec((blk, x.shape[1]), lambda i: (i, 0)),
                  pl.BlockSpec(W.shape, lambda i: (0, 0)),
                  pl.BlockSpec(As.shape, lambda i: (0, 0)),
                  pl.BlockSpec(Ad.shape, lambda i: (0, 0))],
        out_specs=[pl.BlockSpec((blk, dh + L), lambda i: (i, 0)),
                   pl.BlockSpec((blk, L), lambda i: (i, 0))],
        out_shape=[jax.ShapeDtypeStruct((n, dh + L), jnp.float32),
                   jax.ShapeDtypeStruct((n, L), jnp.float32)],
    )(x, W, As, Ad)


def _tc_mid(p0, p1, R, b1, W2, As, Ad):
    n, wd = p0.shape
    d1 = wd - L
    d2 = W2.shape[1]
    blk = _blk(n)

    def body(p0_ref, p1_ref, r_ref, b_ref, w_ref, a_ref, c_ref, o_ref, t_ref):
        sv = p0_ref[...] + p1_ref[...]
        den = jnp.dot(sv[:, d1:], r_ref[...], preferred_element_type=jnp.float32)
        hin = sv[:, :d1] / (den + 1e-16) + b_ref[...]
        hin = jnp.where(hin > 0, hin, jnp.exp(hin) - 1.0)
        h2 = jnp.dot(hin, w_ref[...], preferred_element_type=jnp.float32)
        o_ref[:, :d2] = h2
        o_ref[:, d2:] = jnp.dot(h2, a_ref[...], preferred_element_type=jnp.float32)
        t_ref[...] = jnp.dot(h2, c_ref[...], preferred_element_type=jnp.float32)

    return pl.pallas_call(
        body,
        grid=(n // blk,),
        in_specs=[pl.BlockSpec((blk, wd), lambda i: (i, 0)),
                  pl.BlockSpec((blk, wd), lambda i: (i, 0)),
                  pl.BlockSpec(R.shape, lambda i: (0, 0)),
                  pl.BlockSpec((1, d1), lambda i: (0, 0)),
                  pl.BlockSpec(W2.shape, lambda i: (0, 0)),
                  pl.BlockSpec(As.shape, lambda i: (0, 0)),
                  pl.BlockSpec(Ad.shape, lambda i: (0, 0))],
        out_specs=[pl.BlockSpec((blk, d2 + L), lambda i: (i, 0)),
                   pl.BlockSpec((blk, L), lambda i: (i, 0))],
        out_shape=[jax.ShapeDtypeStruct((n, d2 + L), jnp.float32),
                   jax.ShapeDtypeStruct((n, L), jnp.float32)],
    )(p0, p1, R, b1, W2, As, Ad)


def _tc_fin(q0, q1, R, b2):
    n, wd = q0.shape
    d2 = wd - L
    blk = _blk(n)

    def body(q0_ref, q1_ref, r_ref, b_ref, o_ref):
        sv = q0_ref[...] + q1_ref[...]
        den = jnp.dot(sv[:, d2:], r_ref[...], preferred_element_type=jnp.float32)
        o_ref[...] = sv[:, :d2] / (den + 1e-16) + b_ref[...]

    return pl.pallas_call(
        body,
        grid=(n // blk,),
        in_specs=[pl.BlockSpec((blk, wd), lambda i: (i, 0)),
                  pl.BlockSpec((blk, wd), lambda i: (i, 0)),
                  pl.BlockSpec(R.shape, lambda i: (0, 0)),
                  pl.BlockSpec((1, d2), lambda i: (0, 0))],
        out_specs=pl.BlockSpec((blk, d2), lambda i: (i, 0)),
        out_shape=jax.ShapeDtypeStruct((n, d2), jnp.float32),
    )(q0, q1, R, b2)


def kernel(x, edge_index, W1, att_src1, att_dst1, b1, W2, att_src2, att_dst2, b2):
    n = x.shape[0]
    e = edge_index.shape[1]
    h1, hid1 = att_src1.shape
    d1 = h1 * hid1
    d2 = W2.shape[1]

    nbt = max(-(-e // (NW * EB)), 6)
    nbt = -(-nbt // 3) * 3
    ept = nbt * EB
    pad = ept * NW - e
    npad = ((n + 1 + NS - 1) // NS) * NS
    src = jnp.concatenate([edge_index[0], jnp.zeros((pad,), jnp.int32)])
    dst = jnp.concatenate(
        [edge_index[1], n + (jnp.arange(pad, dtype=jnp.int32) % (npad - n))])

    eye1 = jnp.eye(h1, L, dtype=jnp.float32)
    As1 = (att_src1[:, :, None] * eye1[:, None, :]).reshape(d1, L)
    Ad1 = (att_dst1[:, :, None] * eye1[:, None, :]).reshape(d1, L)
    eye2 = jnp.eye(1, L, dtype=jnp.float32)
    As2 = (att_src2[:, :, None] * eye2[:, None, :]).reshape(d2, L)
    Ad2 = (att_dst2[:, :, None] * eye2[:, None, :]).reshape(d2, L)
    R1 = jnp.repeat(jnp.eye(L, h1, dtype=jnp.float32), d1 // h1, axis=1)
    R2 = jnp.repeat(jnp.eye(L, 1, dtype=jnp.float32), d2, axis=1)

    tot = 2 * nbt

    def _split(frac0):
        return min(max(6, 3 * round(tot * frac0 / 3)), tot - 6)

    ha, t1 = _tc_pre(x, W1, As1, Ad1)
    p = _edge_pass(src, dst, ha, t1, h1, _split(0.60))
    h2a, t2 = _tc_mid(p[0, :n], p[1, :n], R1, b1.reshape(1, d1), W2, As2, Ad2)
    q = _edge_pass(src, dst, h2a, t2, 1, _split(0.62))
    return _tc_fin(q[0, :n], q[1, :n], R2, b2.reshape(1, d2))
